# Initial kernel scaffold; baseline (speedup 1.0000x reference)
import jax
import jax.numpy as jnp
from jax.experimental import pallas as pl


def kernel(inputs, Wp, bp, embed):
    raise NotImplementedError("write your pallas kernel here")


# speedup vs baseline: 6.9081x; 6.9081x over previous
import math

import jax
import jax.numpy as jnp
from jax.experimental import pallas as pl
from jax.experimental.pallas import tpu as pltpu
from jax.experimental.pallas import tpu_sc as plsc

B, CIN, H, W = 4, 384, 32, 32
N = B * H * W
CE = 256
K = 8192
NR = 256
NCHUNK = 1024
NUM_CHUNKS = K // NCHUNK
NUM_R = N // NR

SC_CORES = 2
SC_SUBCORES = 16
NW = SC_CORES * SC_SUBCORES
ROWS_PER_W = N // NW
LANES = 16


def _tc_assign_body(a_ref, wp_ref, bp_ref, e_ref, flat_ref, qidx_ref, he2_ref):
    r = pl.program_id(0)

    @pl.when(r == 0)
    def _():
        for c in range(NUM_CHUNKS):
            e = e_ref[c * NCHUNK:(c + 1) * NCHUNK, :]
            he2 = jax.lax.dot_general(
                jnp.ones((1, CE), jnp.float32), e * e,
                (((1,), (1,)), ((), ())),
                preferred_element_type=jnp.float32,
                precision=jax.lax.Precision.HIGHEST)
            he2_ref[:, c * NCHUNK:(c + 1) * NCHUNK] = 0.5 * he2

    f = jax.lax.dot_general(
        a_ref[...], wp_ref[...], (((1,), (1,)), ((), ())),
        preferred_element_type=jnp.float32) + bp_ref[...]
    flat_ref[...] = f

    best_val = jnp.full((NR, 1), jnp.inf, jnp.float32)
    best_idx = jnp.zeros((NR, 1), jnp.int32)
    for c in range(NUM_CHUNKS):
        e = e_ref[c * NCHUNK:(c + 1) * NCHUNK, :]
        s = jax.lax.dot_general(
            f, e, (((1,), (1,)), ((), ())),
            preferred_element_type=jnp.float32)
        d = he2_ref[:, c * NCHUNK:(c + 1) * NCHUNK] - s
        bmin = jnp.min(d, axis=1, keepdims=True)
        lanes = jax.lax.broadcasted_iota(jnp.int32, (NR, NCHUNK), 1)
        bidx = jnp.min(jnp.where(d == bmin, lanes, jnp.int32(2**30)),
                       axis=1, keepdims=True) + c * NCHUNK
        upd = bmin < best_val
        best_val = jnp.where(upd, bmin, best_val)
        best_idx = jnp.where(upd, bidx, best_idx)
    qidx_ref[...] = best_idx


def _tc_assign(a, wp, bp2, embed):
    return pl.pallas_call(
        _tc_assign_body,
        grid=(NUM_R,),
        in_specs=[
            pl.BlockSpec((NR, CIN), lambda r: (r, 0)),
            pl.BlockSpec((CE, CIN), lambda r: (0, 0)),
            pl.BlockSpec((1, CE), lambda r: (0, 0)),
            pl.BlockSpec((K, CE), lambda r: (0, 0)),
        ],
        out_specs=[
            pl.BlockSpec((NR, CE), lambda r: (r, 0)),
            pl.BlockSpec((NR, 1), lambda r: (r, 0)),
        ],
        out_shape=[
            jax.ShapeDtypeStruct((N, CE), jnp.float32),
            jax.ShapeDtypeStruct((N, 1), jnp.int32),
        ],
        scratch_shapes=[pltpu.VMEM((1, K), jnp.float32)],
        compiler_params=pltpu.CompilerParams(
            dimension_semantics=("arbitrary",)),
    )(a, wp, bp2, embed)


def _sc_gather_body(embed_hbm, idx_hbm, flat_hbm,
                    zq_hbm, psq_hbm, phist_hbm,
                    idx_v, rows_v, flat_v, hist_v, psq_v, sem):
    wid = jax.lax.axis_index("s") * SC_CORES + jax.lax.axis_index("c")
    base = wid * ROWS_PER_W

    pltpu.sync_copy(idx_hbm.at[pl.ds(base, ROWS_PER_W)], idx_v)
    pltpu.async_copy(embed_hbm.at[idx_v], rows_v, sem).wait()
    pltpu.sync_copy(rows_v, zq_hbm.at[pl.ds(base, ROWS_PER_W)])

    pltpu.sync_copy(flat_hbm.at[pl.ds(base, ROWS_PER_W)], flat_v)

    def row_body(i, acc):
        for k in range(CE // LANES):
            dv = (rows_v[i, pl.ds(k * LANES, LANES)]
                  - flat_v[i, pl.ds(k * LANES, LANES)])
            acc = acc + dv * dv
        return acc

    psq_v[...] = jax.lax.fori_loop(
        0, ROWS_PER_W, row_body, jnp.zeros((LANES,), jnp.float32))
    pltpu.sync_copy(psq_v, psq_hbm.at[wid])

    def zero_body(i, carry):
        hist_v[pl.ds(i * LANES, LANES)] = jnp.zeros((LANES,), jnp.float32)
        return carry

    jax.lax.fori_loop(0, K // LANES, zero_body, 0)
    ones = jnp.ones((LANES,), jnp.float32)
    for k in range(ROWS_PER_W // LANES):
        idx_chunk = idx_v[pl.ds(k * LANES, LANES)]
        plsc.addupdate_scatter(hist_v, [idx_chunk], ones)
    pltpu.sync_copy(hist_v, phist_hbm.at[wid])


def _sc_gather(embed, qidx, flat):
    mesh = plsc.VectorSubcoreMesh(core_axis_name="c", subcore_axis_name="s")
    return pl.kernel(
        _sc_gather_body,
        mesh=mesh,
        out_type=[
            jax.ShapeDtypeStruct((N, CE), jnp.float32),
            jax.ShapeDtypeStruct((NW, LANES), jnp.float32),
            jax.ShapeDtypeStruct((NW, K), jnp.float32),
        ],
        scratch_types=[
            pltpu.VMEM((ROWS_PER_W,), jnp.int32),
            pltpu.VMEM((ROWS_PER_W, CE), jnp.float32),
            pltpu.VMEM((ROWS_PER_W, CE), jnp.float32),
            pltpu.VMEM((K,), jnp.float32),
            pltpu.VMEM((LANES,), jnp.float32),
            pltpu.SemaphoreType.DMA,
        ],
        compiler_params=pltpu.CompilerParams(needs_layout_passes=False),
    )(embed, qidx, flat)


def _tc_epilogue_body(psq_ref, phist_ref, loss_ref, lp_ref):
    total = jnp.sum(psq_ref[...])
    loss_ref[...] = (1.25 * (total * (1.0 / float(N * CE)))).reshape(1, 1)
    hist = jnp.sum(phist_ref[...], axis=0, keepdims=True)
    p = hist * (1.0 / float(N))
    lp_ref[...] = (-jnp.sum(p * jnp.log(p + 1e-10))).reshape(1, 1)


def _tc_epilogue(psq, phist):
    return pl.pallas_call(
        _tc_epilogue_body,
        out_shape=[
            jax.ShapeDtypeStruct((1, 1), jnp.float32),
            jax.ShapeDtypeStruct((1, 1), jnp.float32),
        ],
    )(psq, phist)


def kernel(inputs, Wp, bp, embed):
    a = inputs.transpose(0, 3, 2, 1).reshape(N, CIN)
    flat, qidx = _tc_assign(a, Wp, bp.reshape(1, CE), embed)
    zq, psq, phist = _sc_gather(embed, qidx.reshape(N), flat)
    loss, lp = _tc_epilogue(psq, phist)
    z_q = zq.reshape(B, W, H, CE).transpose(0, 3, 2, 1)
    kldiv = jnp.full((B, 1), math.log(K) * float(N // B), jnp.float32)
    return (z_q, loss[0, 0], kldiv, lp[0, 0])


# speedup vs baseline: 9.6518x; 1.3972x over previous
import math

import jax
import jax.numpy as jnp
from jax.experimental import pallas as pl
from jax.experimental.pallas import tpu as pltpu
from jax.experimental.pallas import tpu_sc as plsc

B, CIN, H, W = 4, 384, 32, 32
P = H * W
N = B * P
CE = 256
K = 8192
NCHUNK = 1024
NUM_CHUNKS = K // NCHUNK

SC_CORES = 2
SC_SUBCORES = 16
NW = SC_CORES * SC_SUBCORES
ROWS_PER_W = N // NW
LANES = 16


def _tc_assign_body(x_ref, wp_ref, bp_ref, e_ref,
                    qidx_ref, zsq_ref, dmin_ref, he2_ref):
    b = pl.program_id(0)

    @pl.when(b == 0)
    def _():
        for c in range(NUM_CHUNKS):
            e = e_ref[c * NCHUNK:(c + 1) * NCHUNK, :]
            he2_ref[c * NCHUNK:(c + 1) * NCHUNK, :] = (
                0.5 * jnp.sum(e * e, axis=1, keepdims=True))

    yt = jax.lax.dot_general(
        wp_ref[...], x_ref[0], (((1,), (0,)), ((), ())),
        preferred_element_type=jnp.float32) + bp_ref[...]
    zsq_ref[...] = jnp.sum(yt * yt).reshape(1, 1, 1)

    best_val = jnp.full((1, P), jnp.inf, jnp.float32)
    best_idx = jnp.zeros((1, P), jnp.int32)
    for c in range(NUM_CHUNKS):
        e = e_ref[c * NCHUNK:(c + 1) * NCHUNK, :]
        s = jax.lax.dot_general(
            e, yt, (((1,), (0,)), ((), ())),
            preferred_element_type=jnp.float32)
        d = he2_ref[c * NCHUNK:(c + 1) * NCHUNK, :] - s
        bmin = jnp.min(d, axis=0, keepdims=True)
        rows = jax.lax.broadcasted_iota(jnp.int32, (NCHUNK, P), 0)
        bidx = jnp.min(jnp.where(d == bmin, rows, jnp.int32(2**30)),
                       axis=0, keepdims=True) + c * NCHUNK
        upd = bmin < best_val
        best_val = jnp.where(upd, bmin, best_val)
        best_idx = jnp.where(upd, bidx, best_idx)
    qidx_ref[...] = best_idx.reshape(1, 1, P)
    dmin_ref[...] = jnp.sum(best_val).reshape(1, 1, 1)


def _tc_assign(x3, wp, bpc, embed):
    return pl.pallas_call(
        _tc_assign_body,
        grid=(B,),
        in_specs=[
            pl.BlockSpec((1, CIN, P), lambda b: (b, 0, 0)),
            pl.BlockSpec((CE, CIN), lambda b: (0, 0)),
            pl.BlockSpec((CE, 1), lambda b: (0, 0)),
            pl.BlockSpec((K, CE), lambda b: (0, 0)),
        ],
        out_specs=[
            pl.BlockSpec((1, 1, P), lambda b: (b, 0, 0)),
            pl.BlockSpec((1, 1, 1), lambda b: (b, 0, 0)),
            pl.BlockSpec((1, 1, 1), lambda b: (b, 0, 0)),
        ],
        out_shape=[
            jax.ShapeDtypeStruct((B, 1, P), jnp.int32),
            jax.ShapeDtypeStruct((B, 1, 1), jnp.float32),
            jax.ShapeDtypeStruct((B, 1, 1), jnp.float32),
        ],
        scratch_shapes=[pltpu.VMEM((K, 1), jnp.float32)],
        compiler_params=pltpu.CompilerParams(
            dimension_semantics=("arbitrary",)),
    )(x3, wp, bpc, embed)


def _sc_gather_body(embed_hbm, idx_hbm, zq_hbm, phist_hbm,
                    idx_v, rows_v, hist_v, sem):
    wid = jax.lax.axis_index("s") * SC_CORES + jax.lax.axis_index("c")
    base = wid * ROWS_PER_W

    pltpu.sync_copy(idx_hbm.at[pl.ds(base, ROWS_PER_W)], idx_v)
    pltpu.async_copy(embed_hbm.at[idx_v], rows_v, sem).wait()
    pltpu.sync_copy(rows_v, zq_hbm.at[pl.ds(base, ROWS_PER_W)])

    def zero_body(i, carry):
        hist_v[pl.ds(i * LANES, LANES)] = jnp.zeros((LANES,), jnp.float32)
        return carry

    jax.lax.fori_loop(0, K // LANES, zero_body, 0)
    ones = jnp.ones((LANES,), jnp.float32)
    for k in range(ROWS_PER_W // LANES):
        idx_chunk = idx_v[pl.ds(k * LANES, LANES)]
        plsc.addupdate_scatter(hist_v, [idx_chunk], ones)
    pltpu.sync_copy(hist_v, phist_hbm.at[wid])


def _sc_gather(embed, qidx):
    mesh = plsc.VectorSubcoreMesh(core_axis_name="c", subcore_axis_name="s")
    return pl.kernel(
        _sc_gather_body,
        mesh=mesh,
        out_type=[
            jax.ShapeDtypeStruct((N, CE), jnp.float32),
            jax.ShapeDtypeStruct((NW, K), jnp.float32),
        ],
        scratch_types=[
            pltpu.VMEM((ROWS_PER_W,), jnp.int32),
            pltpu.VMEM((ROWS_PER_W, CE), jnp.float32),
            pltpu.VMEM((K,), jnp.float32),
            pltpu.SemaphoreType.DMA,
        ],
        compiler_params=pltpu.CompilerParams(needs_layout_passes=False),
    )(embed, qidx)


def _tc_epilogue_body(zsq_ref, dmin_ref, phist_ref, loss_ref, lp_ref):
    sq_total = 2.0 * jnp.sum(dmin_ref[...]) + jnp.sum(zsq_ref[...])
    loss_ref[...] = (1.25 * (sq_total * (1.0 / float(N * CE)))).reshape(1, 1)
    hist = jnp.sum(phist_ref[...], axis=0, keepdims=True)
    p = hist * (1.0 / float(N))
    lp_ref[...] = (-jnp.sum(p * jnp.log(p + 1e-10))).reshape(1, 1)


def _tc_epilogue(zsq, dmin, phist):
    return pl.pallas_call(
        _tc_epilogue_body,
        out_shape=[
            jax.ShapeDtypeStruct((1, 1), jnp.float32),
            jax.ShapeDtypeStruct((1, 1), jnp.float32),
        ],
    )(zsq, dmin, phist)


def kernel(inputs, Wp, bp, embed):
    x3 = inputs.reshape(B, CIN, P)
    qidx, zsq, dmin = _tc_assign(x3, Wp, bp.reshape(CE, 1), embed)
    zq, phist = _sc_gather(embed, qidx.reshape(N))
    loss, lp = _tc_epilogue(zsq, dmin, phist)
    z_q = zq.reshape(B, H, W, CE).transpose(0, 3, 1, 2)
    kldiv = jnp.full((B, 1), math.log(K) * float(P), jnp.float32)
    return (z_q, loss[0, 0], kldiv, lp[0, 0])
